# baseline (speedup 1.0000x reference)
import functools

import jax
import jax.numpy as jnp
from jax.experimental import pallas as pl
from jax.experimental.pallas import tpu as pltpu

N = 1024
H = 96
W = 96
TANFOV = 0.5
FX = W / (2.0 * TANFOV)
FY = H / (2.0 * TANFOV)
CAM_Z = 5.0
FEAT = 8

NPIX = H * W
LANES = 128
ROWS = NPIX // LANES
TILE_ROWS = 24
NTILES = ROWS // TILE_ROWS
NPARAM = 7 + 3 + FEAT


def _project_kernel(m3_ref, op_ref, col_ref, feat_ref, sc_ref, rot_ref,
                    table_ref, radii_ref):
    f32 = jnp.float32
    tx = m3_ref[0:1, :]
    ty = m3_ref[1:2, :]
    tz = jnp.maximum(m3_ref[2:3, :] + CAM_Z, 0.2)
    lim = 1.3 * TANFOV
    txc = jnp.clip(tx / tz, -lim, lim) * tz
    tyc = jnp.clip(ty / tz, -lim, lim) * tz
    px = ((tx / tz) / TANFOV + 1.0) * (W - 1) * 0.5
    py = ((ty / tz) / TANFOV + 1.0) * (H - 1) * 0.5

    q0 = rot_ref[0:1, :]
    q1 = rot_ref[1:2, :]
    q2 = rot_ref[2:3, :]
    q3 = rot_ref[3:4, :]
    qn = jnp.sqrt((q0 * q0 + q1 * q1) + (q2 * q2 + q3 * q3))
    r_ = q0 / qn
    x_ = q1 / qn
    y_ = q2 / qn
    z_ = q3 / qn
    R00 = 1 - 2 * (y_ * y_ + z_ * z_)
    R01 = 2 * (x_ * y_ - r_ * z_)
    R02 = 2 * (x_ * z_ + r_ * y_)
    R10 = 2 * (x_ * y_ + r_ * z_)
    R11 = 1 - 2 * (x_ * x_ + z_ * z_)
    R12 = 2 * (y_ * z_ - r_ * x_)
    R20 = 2 * (x_ * z_ - r_ * y_)
    R21 = 2 * (y_ * z_ + r_ * x_)
    R22 = 1 - 2 * (x_ * x_ + y_ * y_)
    s0 = sc_ref[0:1, :]
    s1 = sc_ref[1:2, :]
    s2 = sc_ref[2:3, :]

    def bf(x):
        return x.astype(jnp.bfloat16).astype(f32)

    M00, M01, M02 = bf(R00 * s0), bf(R01 * s1), bf(R02 * s2)
    M10, M11, M12 = bf(R10 * s0), bf(R11 * s1), bf(R12 * s2)
    M20, M21, M22 = bf(R20 * s0), bf(R21 * s1), bf(R22 * s2)
    S00 = bf((M00 * M00 + M01 * M01) + M02 * M02)
    S01 = bf((M00 * M10 + M01 * M11) + M02 * M12)
    S02 = bf((M00 * M20 + M01 * M21) + M02 * M22)
    S11 = bf((M10 * M10 + M11 * M11) + M12 * M12)
    S12 = bf((M10 * M20 + M11 * M21) + M12 * M22)
    S22 = bf((M20 * M20 + M21 * M21) + M22 * M22)

    j00 = bf(FX / tz)
    j02 = bf(-FX * txc / (tz * tz))
    j11 = bf(FY / tz)
    j12 = bf(-FY * tyc / (tz * tz))

    t00 = bf(S00 * j00 + S02 * j02)
    t10 = bf(S01 * j00 + S12 * j02)
    t20 = bf(S02 * j00 + S22 * j02)
    t01 = bf(S01 * j11 + S02 * j12)
    t11 = bf(S11 * j11 + S12 * j12)
    t21 = bf(S12 * j11 + S22 * j12)
    cov00 = t00 * j00 + t20 * j02
    cov01 = t10 * j11 + t20 * j12
    cov11 = t11 * j11 + t21 * j12

    a = cov00 + 0.3
    b = cov01
    c = cov11 + 0.3
    det = jnp.maximum(a * c - b * b, 1e-6)
    conA = c / det
    conB = b / det
    conC = a / det
    mid = 0.5 * (a + c)
    lam1 = mid + jnp.sqrt(jnp.maximum(mid * mid - det, 0.1))
    radii_ref[...] = jnp.ceil(3.0 * jnp.sqrt(jnp.maximum(lam1, 0.0))
                              ).astype(jnp.int32)

    tz_row = jnp.broadcast_to(tz, (N, N))
    tz_col = tz_row.T
    iota_j = jax.lax.broadcasted_iota(jnp.int32, (N, N), 1)
    iota_i = jax.lax.broadcasted_iota(jnp.int32, (N, N), 0)
    before = jnp.where(
        (tz_row < tz_col) | ((tz_row == tz_col) & (iota_j < iota_i)),
        jnp.int32(1), jnp.int32(0))
    rank = jnp.sum(before, axis=1, keepdims=True)

    mask = rank == iota_j

    op_ = op_ref[0:1, :]
    vtab = jnp.concatenate(
        [px, py, conA, conB, conC, op_, tz,
         col_ref[...], feat_ref[...]], axis=0)
    vtab_pad = jnp.concatenate(
        [vtab, jnp.zeros((32 - NPARAM, N), f32)], axis=0)
    tcol = vtab_pad.T
    rows = []
    for c in range(NPARAM):
        masked = jnp.where(mask, tcol[:, c:c + 1], f32(0.0))
        rows.append(jnp.sum(masked, axis=0, keepdims=True))
    table_ref[...] = jnp.concatenate(rows, axis=0)


def _composite_kernel(table_ref, color_ref, depth_ref, weight_ref, feat_ref,
                      w_ref):
    f32 = jnp.float32
    idx = (jax.lax.broadcasted_iota(jnp.int32, (ROWS, LANES), 0) * LANES
           + jax.lax.broadcasted_iota(jnp.int32, (ROWS, LANES), 1))
    xg = (idx % W).astype(f32)
    yg = (idx // W).astype(f32)

    def body1(r, carry):
        T, wt = carry
        dx = xg - table_ref[0, r]
        dy = yg - table_ref[1, r]
        A = table_ref[2, r]
        B = table_ref[3, r]
        C = table_ref[4, r]
        power = jnp.minimum(
            -0.5 * (A * dx * dx + C * dy * dy) - B * dx * dy, 0.0)
        alpha = jnp.minimum(0.99, table_ref[5, r] * jnp.exp(power))
        alpha = jnp.where(alpha < 1.0 / 255.0, 0.0, alpha)
        w = alpha * T
        w_ref[r, :, :] = w
        return (T * (1.0 - alpha), wt + w)

    _, wt = jax.lax.fori_loop(
        0, N, body1,
        (jnp.ones((ROWS, LANES), f32), jnp.zeros((ROWS, LANES), f32)),
        unroll=4)
    weight_ref[...] = wt

    zero = jnp.zeros((ROWS, LANES), f32)

    def body_cd(r, carry):
        c0, c1, c2, d = carry
        w = w_ref[r, :, :]
        c0 = c0 + w * table_ref[7, r]
        c1 = c1 + w * table_ref[8, r]
        c2 = c2 + w * table_ref[9, r]
        d = d + w * table_ref[6, r]
        return (c0, c1, c2, d)

    c0, c1, c2, d = jax.lax.fori_loop(
        0, N, body_cd, (zero, zero, zero, zero), unroll=4)
    color_ref[0, :, :] = c0
    color_ref[1, :, :] = c1
    color_ref[2, :, :] = c2
    depth_ref[...] = d

    def make_feat_body(k0):
        def body_f(r, carry):
            w = w_ref[r, :, :]
            return tuple(carry[k] + w * table_ref[10 + k0 + k, r]
                         for k in range(4))
        return body_f

    for k0 in (0, 4):
        f4 = jax.lax.fori_loop(0, N, make_feat_body(k0),
                               (zero, zero, zero, zero), unroll=4)
        for k in range(4):
            feat_ref[k0 + k, :, :] = f4[k]


@functools.partial(jax.jit, static_argnames=())
def kernel(means3D, means2D, opacities, colors_precomp, features_precomp,
           scales, rotations):
    del means2D
    f32 = jnp.float32
    m3 = means3D.T.astype(f32)
    op_ = opacities.T.astype(f32)
    col = colors_precomp.T.astype(f32)
    feat = features_precomp.T.astype(f32)
    sc = scales.T.astype(f32)
    rot = rotations.T.astype(f32)

    table, radii2d = pl.pallas_call(
        _project_kernel,
        out_shape=(
            jax.ShapeDtypeStruct((NPARAM, N), f32),
            jax.ShapeDtypeStruct((1, N), jnp.int32),
        ),
    )(m3, op_, col, feat, sc, rot)

    color, depth, weight, feature = pl.pallas_call(
        _composite_kernel,
        in_specs=[pl.BlockSpec(memory_space=pltpu.SMEM)],
        out_shape=(
            jax.ShapeDtypeStruct((3, ROWS, LANES), f32),
            jax.ShapeDtypeStruct((ROWS, LANES), f32),
            jax.ShapeDtypeStruct((ROWS, LANES), f32),
            jax.ShapeDtypeStruct((FEAT, ROWS, LANES), f32),
        ),
        scratch_shapes=[pltpu.VMEM((N, ROWS, LANES), f32)],
    )(table)

    return (color.reshape(3, H, W), radii2d.reshape(N),
            depth.reshape(H, W), weight.reshape(H, W),
            feature.reshape(FEAT, H, W))


# speedup vs baseline: 1.0862x; 1.0862x over previous
import functools

import jax
import jax.numpy as jnp
from jax.experimental import pallas as pl
from jax.experimental.pallas import tpu as pltpu

N = 1024
H = 96
W = 96
TANFOV = 0.5
FX = W / (2.0 * TANFOV)
FY = H / (2.0 * TANFOV)
CAM_Z = 5.0
FEAT = 8

NPIX = H * W
LANES = 128
ROWS = NPIX // LANES
TILE_ROWS = 24
NTILES = ROWS // TILE_ROWS
NPARAM = 7 + 3 + FEAT


def _project_kernel(m3_ref, op_ref, col_ref, feat_ref, sc_ref, rot_ref,
                    table_ref, order_ref, radii_ref):
    f32 = jnp.float32
    tx = m3_ref[0:1, :]
    ty = m3_ref[1:2, :]
    tz = jnp.maximum(m3_ref[2:3, :] + CAM_Z, 0.2)
    lim = 1.3 * TANFOV
    txc = jnp.clip(tx / tz, -lim, lim) * tz
    tyc = jnp.clip(ty / tz, -lim, lim) * tz
    px = ((tx / tz) / TANFOV + 1.0) * (W - 1) * 0.5
    py = ((ty / tz) / TANFOV + 1.0) * (H - 1) * 0.5

    q0 = rot_ref[0:1, :]
    q1 = rot_ref[1:2, :]
    q2 = rot_ref[2:3, :]
    q3 = rot_ref[3:4, :]
    qn = jnp.sqrt((q0 * q0 + q1 * q1) + (q2 * q2 + q3 * q3))
    r_ = q0 / qn
    x_ = q1 / qn
    y_ = q2 / qn
    z_ = q3 / qn
    R00 = 1 - 2 * (y_ * y_ + z_ * z_)
    R01 = 2 * (x_ * y_ - r_ * z_)
    R02 = 2 * (x_ * z_ + r_ * y_)
    R10 = 2 * (x_ * y_ + r_ * z_)
    R11 = 1 - 2 * (x_ * x_ + z_ * z_)
    R12 = 2 * (y_ * z_ - r_ * x_)
    R20 = 2 * (x_ * z_ - r_ * y_)
    R21 = 2 * (y_ * z_ + r_ * x_)
    R22 = 1 - 2 * (x_ * x_ + y_ * y_)
    s0 = sc_ref[0:1, :]
    s1 = sc_ref[1:2, :]
    s2 = sc_ref[2:3, :]

    def bf(x):
        return x.astype(jnp.bfloat16).astype(f32)

    M00, M01, M02 = bf(R00 * s0), bf(R01 * s1), bf(R02 * s2)
    M10, M11, M12 = bf(R10 * s0), bf(R11 * s1), bf(R12 * s2)
    M20, M21, M22 = bf(R20 * s0), bf(R21 * s1), bf(R22 * s2)
    S00 = bf((M00 * M00 + M01 * M01) + M02 * M02)
    S01 = bf((M00 * M10 + M01 * M11) + M02 * M12)
    S02 = bf((M00 * M20 + M01 * M21) + M02 * M22)
    S11 = bf((M10 * M10 + M11 * M11) + M12 * M12)
    S12 = bf((M10 * M20 + M11 * M21) + M12 * M22)
    S22 = bf((M20 * M20 + M21 * M21) + M22 * M22)

    j00 = bf(FX / tz)
    j02 = bf(-FX * txc / (tz * tz))
    j11 = bf(FY / tz)
    j12 = bf(-FY * tyc / (tz * tz))

    t00 = bf(S00 * j00 + S02 * j02)
    t10 = bf(S01 * j00 + S12 * j02)
    t20 = bf(S02 * j00 + S22 * j02)
    t01 = bf(S01 * j11 + S02 * j12)
    t11 = bf(S11 * j11 + S12 * j12)
    t21 = bf(S12 * j11 + S22 * j12)
    cov00 = t00 * j00 + t20 * j02
    cov01 = t10 * j11 + t20 * j12
    cov11 = t11 * j11 + t21 * j12

    a = cov00 + 0.3
    b = cov01
    c = cov11 + 0.3
    det = jnp.maximum(a * c - b * b, 1e-6)
    conA = c / det
    conB = b / det
    conC = a / det
    mid = 0.5 * (a + c)
    lam1 = mid + jnp.sqrt(jnp.maximum(mid * mid - det, 0.1))
    radii_ref[...] = jnp.ceil(3.0 * jnp.sqrt(jnp.maximum(lam1, 0.0))
                              ).astype(jnp.int32)

    tz_row = jnp.broadcast_to(tz, (N, N))
    tz_col = tz_row.T
    iota_j = jax.lax.broadcasted_iota(jnp.int32, (N, N), 1)
    iota_i = jax.lax.broadcasted_iota(jnp.int32, (N, N), 0)
    before = jnp.where(
        (tz_row < tz_col) | ((tz_row == tz_col) & (iota_j < iota_i)),
        jnp.int32(1), jnp.int32(0))
    rank = jnp.sum(before, axis=1, keepdims=True)

    onehot = jnp.where(rank == iota_j, jnp.int32(1), jnp.int32(0))
    order_ref[...] = jnp.sum(onehot * iota_i, axis=0, keepdims=True)

    op_ = op_ref[0:1, :]
    table_ref[...] = jnp.concatenate(
        [px, py, conA, conB, conC, op_, tz,
         col_ref[...], feat_ref[...]], axis=0)


def _composite_kernel(table_ref, order_ref, color_ref, depth_ref, weight_ref,
                      feat_ref):
    f32 = jnp.float32
    t = pl.program_id(0)
    p0 = t * (TILE_ROWS * LANES)
    idx = (jax.lax.broadcasted_iota(jnp.int32, (TILE_ROWS, LANES), 0) * LANES
           + jax.lax.broadcasted_iota(jnp.int32, (TILE_ROWS, LANES), 1) + p0)
    xg = (idx % W).astype(f32)
    yg = (idx // W).astype(f32)

    zero = jnp.zeros((TILE_ROWS, LANES), f32)
    init = (jnp.ones((TILE_ROWS, LANES), f32),
            zero, zero, zero,
            zero, zero,
            (zero,) * FEAT)

    def body(r, carry):
        T, c0, c1, c2, d, wt, fs = carry
        g = order_ref[0, r]
        pxg = table_ref[0, g]
        pyg = table_ref[1, g]
        A = table_ref[2, g]
        B = table_ref[3, g]
        C = table_ref[4, g]
        opg = table_ref[5, g]
        tzg = table_ref[6, g]
        dx = xg - pxg
        dy = yg - pyg
        power = jnp.minimum(
            -0.5 * (A * dx * dx + C * dy * dy) - B * dx * dy, 0.0)
        alpha = jnp.minimum(0.99, opg * jnp.exp(power))
        alpha = jnp.where(alpha < 1.0 / 255.0, 0.0, alpha)
        w = alpha * T
        c0 = c0 + w * table_ref[7, g]
        c1 = c1 + w * table_ref[8, g]
        c2 = c2 + w * table_ref[9, g]
        d = d + w * tzg
        wt = wt + w
        fs = tuple(fs[k] + w * table_ref[10 + k, g] for k in range(FEAT))
        T = T * (1.0 - alpha)
        return (T, c0, c1, c2, d, wt, fs)

    T, c0, c1, c2, d, wt, fs = jax.lax.fori_loop(0, N, body, init, unroll=8)
    color_ref[0, :, :] = c0
    color_ref[1, :, :] = c1
    color_ref[2, :, :] = c2
    depth_ref[...] = d
    weight_ref[...] = wt
    for k in range(FEAT):
        feat_ref[k, :, :] = fs[k]


@functools.partial(jax.jit, static_argnames=())
def kernel(means3D, means2D, opacities, colors_precomp, features_precomp,
           scales, rotations):
    del means2D
    f32 = jnp.float32
    m3 = means3D.T.astype(f32)
    op_ = opacities.T.astype(f32)
    col = colors_precomp.T.astype(f32)
    feat = features_precomp.T.astype(f32)
    sc = scales.T.astype(f32)
    rot = rotations.T.astype(f32)

    table, order, radii2d = pl.pallas_call(
        _project_kernel,
        out_shape=(
            jax.ShapeDtypeStruct((NPARAM, N), f32),
            jax.ShapeDtypeStruct((1, N), jnp.int32),
            jax.ShapeDtypeStruct((1, N), jnp.int32),
        ),
    )(m3, op_, col, feat, sc, rot)

    color, depth, weight, feature = pl.pallas_call(
        _composite_kernel,
        grid=(NTILES,),
        in_specs=[pl.BlockSpec(memory_space=pltpu.SMEM),
                  pl.BlockSpec(memory_space=pltpu.SMEM)],
        out_specs=(
            pl.BlockSpec((3, TILE_ROWS, LANES), lambda t: (0, t, 0)),
            pl.BlockSpec((TILE_ROWS, LANES), lambda t: (t, 0)),
            pl.BlockSpec((TILE_ROWS, LANES), lambda t: (t, 0)),
            pl.BlockSpec((FEAT, TILE_ROWS, LANES), lambda t: (0, t, 0)),
        ),
        out_shape=(
            jax.ShapeDtypeStruct((3, ROWS, LANES), f32),
            jax.ShapeDtypeStruct((ROWS, LANES), f32),
            jax.ShapeDtypeStruct((ROWS, LANES), f32),
            jax.ShapeDtypeStruct((FEAT, ROWS, LANES), f32),
        ),
    )(table, order)

    return (color.reshape(3, H, W), radii2d.reshape(N),
            depth.reshape(H, W), weight.reshape(H, W),
            feature.reshape(FEAT, H, W))
